# baseline (speedup 1.0000x reference)
import jax
import jax.numpy as jnp
from jax.experimental import pallas as pl
from jax.experimental.pallas import tpu as pltpu

_ALPHA = 0.25
_IGNORE_INDEX = -100

_ROWS = 1024
_COLS = 100000
_RBLK = 256
_NSPLIT = 4
_CSUB = 1024
_CHUNKS = _CSUB // 128
_CSTEP = _NSPLIT * _CSUB
_NJ = _COLS // _CSTEP + 1
_NCOLBLK = (_COLS + _CSUB - 1) // _CSUB

_LOG2E = 1.4426950408889634
_LN2 = 0.6931471805599453


def _focal_kernel(*refs):
    x_refs = refs[:_NSPLIT]
    tgt_ref, out_ref, m_ref, s_ref, t_ref = refs[_NSPLIT:]
    i = pl.program_id(0)
    j = pl.program_id(1)

    @pl.when(j == 0)
    def _init():
        m_ref[...] = jnp.full((_RBLK, 128), -jnp.inf, jnp.float32)
        s_ref[...] = jnp.zeros((_RBLK, 128), jnp.float32)
        t_ref[...] = jnp.zeros((_RBLK, 128), jnp.float32)

    tgt = tgt_ref[...]
    lane = jax.lax.broadcasted_iota(jnp.int32, (_RBLK, 128), 1)
    jbase = j * _CSTEP

    def process(chunks):
        bm = None
        for k, c, masked in chunks:
            y = x_refs[k][:, c * 128:(c + 1) * 128] * _LOG2E
            if masked:
                col = jbase + (k * _CSUB + c * 128) + lane
                y = jnp.where(col < _COLS, y, -jnp.inf)
            bm = y if bm is None else jnp.maximum(bm, y)
        m_old = m_ref[...]
        m_new = jnp.maximum(m_old, bm)
        s = s_ref[...] * jnp.exp2(m_old - m_new)
        t = t_ref[...]
        for k, c, masked in chunks:
            y = x_refs[k][:, c * 128:(c + 1) * 128] * _LOG2E
            col = jbase + (k * _CSUB + c * 128) + lane
            if masked:
                y = jnp.where(col < _COLS, y, -jnp.inf)
            s = s + jnp.exp2(y - m_new)
            t = t + jnp.where(col == tgt, y, 0.0)
        m_ref[...] = m_new
        s_ref[...] = s
        t_ref[...] = t
        return m_new, s, t

    is_last = j == _NJ - 1

    @pl.when(jnp.logical_not(is_last))
    def _full_step():
        process([(k, c, False) for k in range(_NSPLIT) for c in range(_CHUNKS)])

    @pl.when(is_last)
    def _last_step():
        base = (_NJ - 1) * _CSTEP
        chunks = []
        for k in range(_NSPLIT):
            for c in range(_CHUNKS):
                start = base + k * _CSUB + c * 128
                if start + 128 <= _COLS:
                    chunks.append((k, c, False))
                elif start < _COLS:
                    chunks.append((k, c, True))
        m_lane, s_lane, t_lane = process(chunks)
        m_row = jnp.max(m_lane, axis=1, keepdims=True)
        s_row = jnp.sum(
            s_lane * jnp.exp2(m_lane - m_row), axis=1, keepdims=True
        )
        t_row = jnp.sum(t_lane, axis=1, keepdims=True)
        nll = _LN2 * (m_row + jnp.log2(s_row) - t_row)
        loss = jnp.where(tgt == _IGNORE_INDEX, 0.0, nll)
        pt = jnp.exp(-loss)
        fl = _ALPHA * (1.0 - pt) * (1.0 - pt) * loss
        out_ref[0, 0, 0] = jnp.sum(fl) * (1.0 / _ROWS)


def _make_index_map(k):
    def index_map(i, j):
        return (i, jnp.minimum(j * _NSPLIT + k, _NCOLBLK - 1))

    return index_map


def kernel(input, target):
    tgt2d = target.astype(jnp.int32).reshape(_ROWS, 1)
    out = pl.pallas_call(
        _focal_kernel,
        grid=(_ROWS // _RBLK, _NJ),
        in_specs=[
            pl.BlockSpec((_RBLK, _CSUB), _make_index_map(k))
            for k in range(_NSPLIT)
        ]
        + [pl.BlockSpec((_RBLK, 1), lambda i, j: (i, 0))],
        out_specs=pl.BlockSpec(
            (1, 1, 1), lambda i, j: (i, 0, 0), memory_space=pltpu.SMEM
        ),
        out_shape=jax.ShapeDtypeStruct((_ROWS // _RBLK, 1, 1), jnp.float32),
        compiler_params=pltpu.CompilerParams(
            dimension_semantics=("parallel", "arbitrary")
        ),
        scratch_shapes=[
            pltpu.VMEM((_RBLK, 128), jnp.float32),
            pltpu.VMEM((_RBLK, 128), jnp.float32),
            pltpu.VMEM((_RBLK, 128), jnp.float32),
        ],
    )(*([input] * _NSPLIT), tgt2d)
    return jnp.sum(out)


# speedup vs baseline: 1.0281x; 1.0281x over previous
import jax
import jax.numpy as jnp
from jax.experimental import pallas as pl
from jax.experimental.pallas import tpu as pltpu

_ALPHA = 0.25
_IGNORE_INDEX = -100

_ROWS = 1024
_COLS = 100000
_RBLK = 256
_NSPLIT = 1
_CSUB = 4096
_CHUNKS = _CSUB // 128
_CSTEP = _NSPLIT * _CSUB
_NJ = _COLS // _CSTEP + 1
_NCOLBLK = (_COLS + _CSUB - 1) // _CSUB

_LOG2E = 1.4426950408889634
_LN2 = 0.6931471805599453


def _focal_kernel(*refs):
    x_refs = refs[:_NSPLIT]
    tgt_ref, out_ref, m_ref, s_ref, t_ref = refs[_NSPLIT:]
    i = pl.program_id(0)
    j = pl.program_id(1)

    @pl.when(j == 0)
    def _init():
        m_ref[...] = jnp.full((_RBLK, 128), -jnp.inf, jnp.float32)
        s_ref[...] = jnp.zeros((_RBLK, 128), jnp.float32)
        t_ref[...] = jnp.zeros((_RBLK, 128), jnp.float32)

    tgt = tgt_ref[...]
    lane = jax.lax.broadcasted_iota(jnp.int32, (_RBLK, 128), 1)
    jbase = j * _CSTEP

    def process(chunks):
        bm = None
        for k, c, masked in chunks:
            y = x_refs[k][:, c * 128:(c + 1) * 128] * _LOG2E
            if masked:
                col = jbase + (k * _CSUB + c * 128) + lane
                y = jnp.where(col < _COLS, y, -jnp.inf)
            bm = y if bm is None else jnp.maximum(bm, y)
        m_old = m_ref[...]
        m_new = jnp.maximum(m_old, bm)
        s = s_ref[...] * jnp.exp2(m_old - m_new)
        t = t_ref[...]
        for k, c, masked in chunks:
            y = x_refs[k][:, c * 128:(c + 1) * 128] * _LOG2E
            col = jbase + (k * _CSUB + c * 128) + lane
            if masked:
                y = jnp.where(col < _COLS, y, -jnp.inf)
            s = s + jnp.exp2(y - m_new)
            t = t + jnp.where(col == tgt, y, 0.0)
        m_ref[...] = m_new
        s_ref[...] = s
        t_ref[...] = t
        return m_new, s, t

    is_last = j == _NJ - 1

    @pl.when(jnp.logical_not(is_last))
    def _full_step():
        process([(k, c, False) for k in range(_NSPLIT) for c in range(_CHUNKS)])

    @pl.when(is_last)
    def _last_step():
        base = (_NJ - 1) * _CSTEP
        chunks = []
        for k in range(_NSPLIT):
            for c in range(_CHUNKS):
                start = base + k * _CSUB + c * 128
                if start + 128 <= _COLS:
                    chunks.append((k, c, False))
                elif start < _COLS:
                    chunks.append((k, c, True))
        m_lane, s_lane, t_lane = process(chunks)
        m_row = jnp.max(m_lane, axis=1, keepdims=True)
        s_row = jnp.sum(
            s_lane * jnp.exp2(m_lane - m_row), axis=1, keepdims=True
        )
        t_row = jnp.sum(t_lane, axis=1, keepdims=True)
        nll = _LN2 * (m_row + jnp.log2(s_row) - t_row)
        loss = jnp.where(tgt == _IGNORE_INDEX, 0.0, nll)
        pt = jnp.exp(-loss)
        fl = _ALPHA * (1.0 - pt) * (1.0 - pt) * loss
        out_ref[0, 0, 0] = jnp.sum(fl) * (1.0 / _ROWS)


def _make_index_map(k):
    def index_map(i, j):
        return (i, jnp.minimum(j * _NSPLIT + k, _NCOLBLK - 1))

    return index_map


def kernel(input, target):
    tgt2d = target.astype(jnp.int32).reshape(_ROWS, 1)
    out = pl.pallas_call(
        _focal_kernel,
        grid=(_ROWS // _RBLK, _NJ),
        in_specs=[
            pl.BlockSpec((_RBLK, _CSUB), _make_index_map(k))
            for k in range(_NSPLIT)
        ]
        + [pl.BlockSpec((_RBLK, 1), lambda i, j: (i, 0))],
        out_specs=pl.BlockSpec(
            (1, 1, 1), lambda i, j: (i, 0, 0), memory_space=pltpu.SMEM
        ),
        out_shape=jax.ShapeDtypeStruct((_ROWS // _RBLK, 1, 1), jnp.float32),
        compiler_params=pltpu.CompilerParams(
            dimension_semantics=("parallel", "arbitrary")
        ),
        scratch_shapes=[
            pltpu.VMEM((_RBLK, 128), jnp.float32),
            pltpu.VMEM((_RBLK, 128), jnp.float32),
            pltpu.VMEM((_RBLK, 128), jnp.float32),
        ],
    )(*([input] * _NSPLIT), tgt2d)
    return jnp.sum(out)


# speedup vs baseline: 1.0581x; 1.0292x over previous
import jax
import jax.numpy as jnp
from jax.experimental import pallas as pl
from jax.experimental.pallas import tpu as pltpu

_ALPHA = 0.25
_IGNORE_INDEX = -100

_ROWS = 1024
_COLS = 100000
_RBLK = 256
_CSUB = 4096
_CHUNKS = _CSUB // 128
_NJ = _COLS // _CSUB + 1
_NCOLBLK = (_COLS + _CSUB - 1) // _CSUB

_LOG2E = 1.4426950408889634
_LN2 = 0.6931471805599453


def _focal_kernel(x_ref, tgt_ref, out_ref, m_ref, s_ref, t_ref):
    i = pl.program_id(0)
    j = pl.program_id(1)

    @pl.when(j == 0)
    def _init():
        m_ref[...] = jnp.full((_RBLK, 128), -jnp.inf, jnp.float32)
        s_ref[...] = jnp.zeros((_RBLK, 128), jnp.float32)
        t_ref[...] = jnp.zeros((_RBLK, 128), jnp.float32)

    tgt = tgt_ref[...]
    lane = jax.lax.broadcasted_iota(jnp.int32, (_RBLK, 128), 1)
    rel_tgt = tgt - j * _CSUB
    rel_end = _COLS - j * _CSUB

    rel_tgt_b = jnp.broadcast_to(rel_tgt, (_RBLK, 128))
    rel_end_b = jnp.broadcast_to(jnp.int32(rel_end), (_RBLK, 128))

    def process(chunks):
        m_old = m_ref[...]
        s = s_ref[...]
        t = t_ref[...]
        for g in range(0, len(chunks), 4):
            group = chunks[g:g + 4]
            bm = None
            for c, masked in group:
                xc = x_ref[:, c * 128:(c + 1) * 128]
                if masked:
                    xc = jnp.where(lane + c * 128 < rel_end_b, xc, -jnp.inf)
                bm = xc if bm is None else jnp.maximum(bm, xc)
            m_new = jnp.maximum(m_old, bm)
            s = s * jnp.exp2((m_old - m_new) * _LOG2E)
            for c, masked in group:
                xc = x_ref[:, c * 128:(c + 1) * 128]
                e = jnp.exp2((xc - m_new) * _LOG2E)
                if masked:
                    e = jnp.where(lane + c * 128 < rel_end_b, e, 0.0)
                s = s + e
                t = jnp.where(lane + c * 128 == rel_tgt_b, xc, t)
            m_old = m_new
        m_ref[...] = m_old
        s_ref[...] = s
        t_ref[...] = t
        return m_old, s, t

    is_last = j == _NJ - 1

    @pl.when(jnp.logical_not(is_last))
    def _full_step():
        process([(c, False) for c in range(_CHUNKS)])

    @pl.when(is_last)
    def _last_step():
        base = (_NJ - 1) * _CSUB
        chunks = []
        for c in range(_CHUNKS):
            start = base + c * 128
            if start + 128 <= _COLS:
                chunks.append((c, False))
            elif start < _COLS:
                chunks.append((c, True))
        m_lane, s_lane, t_lane = process(chunks)
        m_row = jnp.max(m_lane, axis=1, keepdims=True)
        s_row = jnp.sum(
            s_lane * jnp.exp2((m_lane - m_row) * _LOG2E),
            axis=1,
            keepdims=True,
        )
        t_row = jnp.sum(t_lane, axis=1, keepdims=True)
        nll = m_row + _LN2 * jnp.log2(s_row) - t_row
        loss = jnp.where(tgt == _IGNORE_INDEX, 0.0, nll)
        pt = jnp.exp(-loss)
        fl = _ALPHA * (1.0 - pt) * (1.0 - pt) * loss
        partial = jnp.sum(fl) * (1.0 / _ROWS)

        @pl.when(i == 0)
        def _zero():
            out_ref[0, 0] = 0.0

        out_ref[0, 0] += partial


def kernel(input, target):
    tgt2d = target.astype(jnp.int32).reshape(_ROWS, 1)
    out = pl.pallas_call(
        _focal_kernel,
        grid=(_ROWS // _RBLK, _NJ),
        in_specs=[
            pl.BlockSpec((_RBLK, _CSUB), lambda i, j: (i, j)),
            pl.BlockSpec((_RBLK, 1), lambda i, j: (i, 0)),
        ],
        out_specs=pl.BlockSpec(
            (1, 1), lambda i, j: (0, 0), memory_space=pltpu.SMEM
        ),
        out_shape=jax.ShapeDtypeStruct((1, 1), jnp.float32),
        scratch_shapes=[
            pltpu.VMEM((_RBLK, 128), jnp.float32),
            pltpu.VMEM((_RBLK, 128), jnp.float32),
            pltpu.VMEM((_RBLK, 128), jnp.float32),
        ],
    )(input, tgt2d)
    return out[0, 0]
